# baseline (speedup 1.0000x reference)
import jax
import jax.numpy as jnp
from jax.experimental import pallas as pl
from jax.experimental.pallas import tpu as pltpu

_B = 512
_F = 4 * 64 * 64
_BB = 32


def _body(t_ref, sa_ref, som_ref, x_ref, n_ref, o_ref):
    i = pl.program_id(0)
    base = i * _BB
    a_list = []
    b_list = []
    for k in range(_BB):
        tk = t_ref[base + k]
        a_list.append(sa_ref[tk])
        b_list.append(som_ref[tk])
    a_col = jnp.stack(a_list).reshape(_BB, 1, 1, 1)
    b_col = jnp.stack(b_list).reshape(_BB, 1, 1, 1)
    o_ref[...] = a_col * x_ref[...] + b_col * n_ref[...]


def kernel(x0, t, noise, sqrt_alphas_cumprod, sqrt_one_minus_alphas_cumprod):
    grid_spec = pltpu.PrefetchScalarGridSpec(
        num_scalar_prefetch=3,
        grid=(_B // _BB,),
        in_specs=[
            pl.BlockSpec((_BB, 4, 64, 64), lambda i, *_: (i, 0, 0, 0)),
            pl.BlockSpec((_BB, 4, 64, 64), lambda i, *_: (i, 0, 0, 0)),
        ],
        out_specs=pl.BlockSpec((_BB, 4, 64, 64), lambda i, *_: (i, 0, 0, 0)),
    )
    out = pl.pallas_call(
        _body,
        grid_spec=grid_spec,
        out_shape=jax.ShapeDtypeStruct((_B, 4, 64, 64), jnp.float32),
        compiler_params=pltpu.CompilerParams(
            dimension_semantics=("parallel",),
        ),
    )(
        t.astype(jnp.int32),
        sqrt_alphas_cumprod,
        sqrt_one_minus_alphas_cumprod,
        x0,
        noise,
    )
    return out


# speedup vs baseline: 1.3611x; 1.3611x over previous
import jax
import jax.numpy as jnp
from jax.experimental import pallas as pl
from jax.experimental.pallas import tpu as pltpu

_B = 512
_R = 4 * 64
_C = 64
_BB = 32


def _body(t_ref, sa_ref, som_ref, x_ref, n_ref, o_ref):
    i = pl.program_id(0)
    base = i * _BB
    a_list = []
    b_list = []
    for k in range(_BB):
        tk = t_ref[base + k]
        a_list.append(sa_ref[tk])
        b_list.append(som_ref[tk])
    a_col = jnp.stack(a_list).reshape(_BB, 1, 1)
    b_col = jnp.stack(b_list).reshape(_BB, 1, 1)
    o_ref[...] = a_col * x_ref[...] + b_col * n_ref[...]


def kernel(x0, t, noise, sqrt_alphas_cumprod, sqrt_one_minus_alphas_cumprod):
    x3 = x0.reshape(_B, _R, _C)
    n3 = noise.reshape(_B, _R, _C)
    grid_spec = pltpu.PrefetchScalarGridSpec(
        num_scalar_prefetch=3,
        grid=(_B // _BB,),
        in_specs=[
            pl.BlockSpec((_BB, _R, _C), lambda i, *_: (i, 0, 0)),
            pl.BlockSpec((_BB, _R, _C), lambda i, *_: (i, 0, 0)),
        ],
        out_specs=pl.BlockSpec((_BB, _R, _C), lambda i, *_: (i, 0, 0)),
    )
    out = pl.pallas_call(
        _body,
        grid_spec=grid_spec,
        out_shape=jax.ShapeDtypeStruct((_B, _R, _C), jnp.float32),
        compiler_params=pltpu.CompilerParams(
            dimension_semantics=("parallel",),
        ),
    )(
        t.astype(jnp.int32),
        sqrt_alphas_cumprod,
        sqrt_one_minus_alphas_cumprod,
        x3,
        n3,
    )
    return out.reshape(x0.shape)


# speedup vs baseline: 5.6845x; 4.1764x over previous
import jax
import jax.numpy as jnp
from jax.experimental import pallas as pl
from jax.experimental.pallas import tpu as pltpu

_B = 512
_R = 4 * 64 * 64
_RB = 2048


def _body(a_ref, b_ref, x_ref, n_ref, o_ref):
    o_ref[...] = a_ref[...] * x_ref[...] + b_ref[...] * n_ref[...]


def kernel(x0, t, noise, sqrt_alphas_cumprod, sqrt_one_minus_alphas_cumprod):
    xT = jnp.transpose(x0, (1, 2, 3, 0)).reshape(_R, _B)
    nT = jnp.transpose(noise, (1, 2, 3, 0)).reshape(_R, _B)
    a_row = jnp.take(sqrt_alphas_cumprod, t, axis=0).reshape(1, _B)
    b_row = jnp.take(sqrt_one_minus_alphas_cumprod, t, axis=0).reshape(1, _B)
    out = pl.pallas_call(
        _body,
        grid=(_R // _RB,),
        in_specs=[
            pl.BlockSpec((1, _B), lambda i: (0, 0)),
            pl.BlockSpec((1, _B), lambda i: (0, 0)),
            pl.BlockSpec((_RB, _B), lambda i: (i, 0)),
            pl.BlockSpec((_RB, _B), lambda i: (i, 0)),
        ],
        out_specs=pl.BlockSpec((_RB, _B), lambda i: (i, 0)),
        out_shape=jax.ShapeDtypeStruct((_R, _B), jnp.float32),
        compiler_params=pltpu.CompilerParams(
            dimension_semantics=("parallel",),
        ),
    )(a_row, b_row, xT, nT)
    return out.reshape(4, 64, 64, _B).transpose(3, 0, 1, 2)
